# baseline (speedup 1.0000x reference)
import functools

import jax
import jax.numpy as jnp
from jax import lax
from jax.experimental import pallas as pl
from jax.experimental.pallas import tpu as pltpu
from jax.experimental.pallas import tpu_sc as plsc

IN_F = 4096
OUT_F = 4096
TOTAL = IN_F * OUT_F

NW = 32
CSZ = 65536
CROWS = CSZ // IN_F
SHIFT = 12
NCH = TOTAL // CSZ
NSUB = NCH // NW
WIN = 8192
WBUF = WIN + 16
SBUF = ((NCH + 1 + 31) // 16) * 16


def _sel(buf, i):
    return buf[pl.ds(i, 16)][0]


def _scatter_body(flat_hbm, vals_hbm, starts_hbm, w_hbm,
                  sbuf, dense, fwin, vwin, sem_ld, sem_o):
    w = lax.axis_index("s") * 2 + lax.axis_index("c")

    pltpu.sync_copy(starts_hbm, sbuf)

    @pl.loop(0, CROWS)
    def _zero_row(r):
        @pl.loop(0, IN_F // 16)
        def _zero_init(i):
            dense[r, pl.ds(i * 16, 16)] = jnp.zeros((16,), jnp.float32)

    def _windows(c, start, end, value_of):
        astart = jnp.bitwise_and(start, jnp.int32(-8))
        delta = start - astart
        cbase = c * CSZ
        nwin = (end - start + WIN - 1) // WIN

        def _win(m, carry):
            off = pl.multiple_of(astart + m * WIN, 8)
            ld1 = pltpu.async_copy(flat_hbm.at[pl.ds(off, WBUF)], fwin,
                                   sem_ld)
            ld2 = pltpu.async_copy(vals_hbm.at[pl.ds(off, WBUF)], vwin,
                                   sem_ld)
            ld1.wait()
            ld2.wait()
            rem = end - start - m * WIN
            n_j = (jnp.minimum(rem, WIN) + 15) // 16
            wbase = start + m * WIN

            @pl.loop(0, n_j)
            def _scat(j):
                t = j * 16
                fv = fwin[pl.ds(delta + t, 16)]
                vv = vwin[pl.ds(delta + t, 16)]
                g = wbase + t + lax.broadcasted_iota(jnp.int32, (16,), 0)
                mask = g < end
                lidx = fv - cbase
                lrow = jax.lax.shift_right_logical(lidx, SHIFT)
                lcol = jnp.bitwise_and(lidx, IN_F - 1)
                plsc.store_scatter(dense, [lrow, lcol], value_of(vv),
                                   mask=mask)

            return carry

        lax.fori_loop(0, nwin, _win, 0)

    @pl.loop(0, NSUB)
    def _sub(s):
        c = w * NSUB + s
        start = _sel(sbuf, c)
        end = _sel(sbuf, c + 1)
        with jax.named_scope("sc_scatter"):
            _windows(c, start, end, lambda v: v)
        with jax.named_scope("sc_dma_out"):
            r0 = pl.multiple_of(c * CROWS, CROWS)
            pltpu.async_copy(dense, w_hbm.at[pl.ds(r0, CROWS)],
                             sem_o).wait()
        with jax.named_scope("sc_clean"):
            _windows(c, start, end,
                     lambda v: jnp.zeros((16,), jnp.float32))


def _densify(flat_p, vals_p, starts):
    mesh = plsc.VectorSubcoreMesh(core_axis_name="c", subcore_axis_name="s")
    return pl.kernel(
        _scatter_body,
        out_type=jax.ShapeDtypeStruct((OUT_F, IN_F), jnp.float32),
        mesh=mesh,
        compiler_params=pltpu.CompilerParams(needs_layout_passes=False),
        scratch_types=[
            pltpu.VMEM((SBUF,), jnp.int32),
            pltpu.VMEM((CROWS, IN_F), jnp.float32),
            pltpu.VMEM((WBUF,), jnp.int32),
            pltpu.VMEM((WBUF,), jnp.float32),
            pltpu.SemaphoreType.DMA,
            pltpu.SemaphoreType.DMA,
        ],
    )(flat_p, vals_p, starts)


def _mm_body(x_ref, w_ref, b_ref, o_ref):
    acc = lax.dot_general(
        x_ref[...], w_ref[...],
        (((1,), (1,)), ((), ())),
        preferred_element_type=jnp.float32)
    o_ref[...] = acc + b_ref[...][None, :]


def _matmul(x, w, bias, batch):
    nb = 512
    return pl.pallas_call(
        _mm_body,
        grid=(OUT_F // nb,),
        in_specs=[
            pl.BlockSpec((batch, IN_F), lambda j: (0, 0)),
            pl.BlockSpec((nb, IN_F), lambda j: (j, 0)),
            pl.BlockSpec((nb,), lambda j: (j,)),
        ],
        out_specs=pl.BlockSpec((batch, nb), lambda j: (0, j)),
        out_shape=jax.ShapeDtypeStruct((batch, OUT_F), jnp.float32),
    )(x, w, bias)


def kernel(x, sparse_indices, sparse_values, bias):
    orig_shape = x.shape
    x2d = x.reshape(-1, IN_F)
    batch = x2d.shape[0]

    nnz = sparse_values.shape[0]
    flat = sparse_indices[0] * IN_F + sparse_indices[1]
    padn = -(-(nnz + 2 * WIN) // 16) * 16
    pad = padn - nnz
    flat_p = jnp.concatenate([flat, jnp.zeros((pad,), flat.dtype)])
    vals_p = jnp.concatenate(
        [sparse_values, jnp.zeros((pad,), sparse_values.dtype)])
    S = 64
    ns = -(-nnz // S)
    pad_hi = jnp.full((ns * S - nnz,), jnp.iinfo(jnp.int32).max, jnp.int32)
    flat_hi = jnp.concatenate([flat, pad_hi])
    sample = jnp.take(flat_hi, jnp.arange(ns, dtype=jnp.int32) * S)
    bounds = jnp.arange(NCH + 1, dtype=flat.dtype) * CSZ
    coarse = jnp.sum(sample[None, :] < bounds[:, None],
                     axis=1).astype(jnp.int32)
    base = jnp.maximum(coarse - 1, 0) * S
    wins = jnp.take(
        flat_hi, base[:, None] + jnp.arange(S, dtype=jnp.int32)[None, :])
    cnts = jnp.sum(wins < bounds[:, None], axis=1).astype(jnp.int32)
    starts = base + cnts
    starts_p = jnp.concatenate(
        [starts, jnp.zeros((SBUF - NCH - 1,), jnp.int32)])

    w = _densify(flat_p, vals_p, starts_p)
    y = _matmul(x2d, w, bias, batch)
    return y.reshape(*orig_shape[:-1], OUT_F).astype(x.dtype)


# speedup vs baseline: 2.4040x; 2.4040x over previous
import functools

import jax
import jax.numpy as jnp
from jax import lax
from jax.experimental import pallas as pl
from jax.experimental.pallas import tpu as pltpu
from jax.experimental.pallas import tpu_sc as plsc

IN_F = 4096
OUT_F = 4096
TOTAL = IN_F * OUT_F

NW = 32
CSZ = 65536
CROWS = CSZ // IN_F
SHIFT = 12
SAMP = 512
NCH = TOTAL // CSZ
NSUB = NCH // NW
WIN = 8192
WBUF = WIN + 16
SBUF = ((NCH + 1 + 31) // 16) * 16


def _sel(buf, i):
    return buf[pl.ds(i, 16)][0]


def _scatter_body(flat_hbm, vals_hbm, starts_hbm, w_hbm,
                  sbuf, dense, fwin, vwin, sem_ld, sem_o):
    w = lax.axis_index("s") * 2 + lax.axis_index("c")

    pltpu.sync_copy(starts_hbm, sbuf)

    @pl.loop(0, CROWS)
    def _zero_row(r):
        @pl.loop(0, IN_F // 16)
        def _zero_init(i):
            dense[r, pl.ds(i * 16, 16)] = jnp.zeros((16,), jnp.float32)

    def _windows(c, start, end, value_of):
        astart = jnp.bitwise_and(start, jnp.int32(-8))
        delta = start - astart
        cbase = c * CSZ
        nwin = (end - start + WIN - 1) // WIN

        def _win(m, carry):
            off = pl.multiple_of(astart + m * WIN, 8)
            ld1 = pltpu.async_copy(flat_hbm.at[pl.ds(off, WBUF)], fwin,
                                   sem_ld)
            ld2 = pltpu.async_copy(vals_hbm.at[pl.ds(off, WBUF)], vwin,
                                   sem_ld)
            ld1.wait()
            ld2.wait()
            rem = end - start - m * WIN
            n_j = (jnp.minimum(rem, WIN) + 15) // 16
            wbase = start + m * WIN

            @pl.loop(0, n_j)
            def _scat(j):
                t = j * 16
                fv = fwin[pl.ds(delta + t, 16)]
                vv = vwin[pl.ds(delta + t, 16)]
                g = wbase + t + lax.broadcasted_iota(jnp.int32, (16,), 0)
                lidx = fv - cbase
                mask = ((g < end) & (lidx >= 0) & (lidx < CSZ))
                lrow = jax.lax.shift_right_logical(lidx, SHIFT)
                lcol = jnp.bitwise_and(lidx, IN_F - 1)
                plsc.store_scatter(dense, [lrow, lcol], value_of(vv),
                                   mask=mask)

            return carry

        lax.fori_loop(0, nwin, _win, 0)

    @pl.loop(0, NSUB)
    def _sub(s):
        c = w * NSUB + s
        co0 = _sel(sbuf, c)
        co1 = _sel(sbuf, c + 1)
        lo = jnp.maximum(co0 - 1, 0) * SAMP
        hi = co1 * SAMP
        with jax.named_scope("sc_scatter"):
            _windows(c, lo, hi, lambda v: v)
        with jax.named_scope("sc_dma_out"):
            r0 = pl.multiple_of(c * CROWS, CROWS)
            pltpu.async_copy(dense, w_hbm.at[pl.ds(r0, CROWS)],
                             sem_o).wait()
        with jax.named_scope("sc_clean"):
            _windows(c, lo, hi,
                     lambda v: jnp.zeros((16,), jnp.float32))


def _densify(flat_p, vals_p, starts):
    mesh = plsc.VectorSubcoreMesh(core_axis_name="c", subcore_axis_name="s")
    return pl.kernel(
        _scatter_body,
        out_type=jax.ShapeDtypeStruct((OUT_F, IN_F), jnp.float32),
        mesh=mesh,
        compiler_params=pltpu.CompilerParams(needs_layout_passes=False),
        scratch_types=[
            pltpu.VMEM((SBUF,), jnp.int32),
            pltpu.VMEM((CROWS, IN_F), jnp.float32),
            pltpu.VMEM((WBUF,), jnp.int32),
            pltpu.VMEM((WBUF,), jnp.float32),
            pltpu.SemaphoreType.DMA,
            pltpu.SemaphoreType.DMA,
        ],
    )(flat_p, vals_p, starts)


def _mm_body(x_ref, w_ref, b_ref, o_ref):
    acc = lax.dot_general(
        x_ref[...], w_ref[...],
        (((1,), (1,)), ((), ())),
        preferred_element_type=jnp.float32)
    o_ref[...] = acc + b_ref[...][None, :]


def _matmul(x, w, bias, batch):
    nb = 512
    return pl.pallas_call(
        _mm_body,
        grid=(OUT_F // nb,),
        in_specs=[
            pl.BlockSpec((batch, IN_F), lambda j: (0, 0)),
            pl.BlockSpec((nb, IN_F), lambda j: (j, 0)),
            pl.BlockSpec((nb,), lambda j: (j,)),
        ],
        out_specs=pl.BlockSpec((batch, nb), lambda j: (0, j)),
        out_shape=jax.ShapeDtypeStruct((batch, OUT_F), jnp.float32),
    )(x, w, bias)


def kernel(x, sparse_indices, sparse_values, bias):
    orig_shape = x.shape
    x2d = x.reshape(-1, IN_F)
    batch = x2d.shape[0]

    nnz = sparse_values.shape[0]
    flat = sparse_indices[0] * IN_F + sparse_indices[1]
    padn = -(-(nnz + 3 * WIN) // SAMP) * SAMP
    pad = padn - nnz
    flat_p = jnp.concatenate(
        [flat, jnp.full((pad,), TOTAL, flat.dtype)])
    vals_p = jnp.concatenate(
        [sparse_values, jnp.zeros((pad,), sparse_values.dtype)])
    ns = -(-nnz // SAMP)
    sample = jnp.min(flat_p[:ns * SAMP].reshape(ns, SAMP), axis=1)
    bounds = jnp.arange(NCH + 1, dtype=flat.dtype) * CSZ
    coarse = jnp.sum(sample[None, :] < bounds[:, None],
                     axis=1).astype(jnp.int32)
    coarse_p = jnp.concatenate(
        [coarse, jnp.zeros((SBUF - NCH - 1,), jnp.int32)])

    w = _densify(flat_p, vals_p, coarse_p)
    y = _matmul(x2d, w, bias, batch)
    return y.reshape(*orig_shape[:-1], OUT_F).astype(x.dtype)
